# baseline (speedup 1.0000x reference)
import functools
import math

import jax
import jax.numpy as jnp
from jax import lax
from jax.experimental import pallas as pl
from jax.experimental.pallas import tpu as pltpu

_BETA = 0.6
_ALPHA = 0.25
_EPS = 1e-5
_E = math.e
_VMEM_LIMIT = 48 * 1024 * 1024


def _pick_rb(n):
    for c in (768, 512, 384, 256, 128):
        if n % c == 0:
            return c
    return n


def _prelu(x):
    return jnp.where(x >= 0.0, x, _ALPHA * x)


def _bf16(x):
    return x.astype(jnp.bfloat16)


def _f32(x):
    return x.astype(jnp.float32)


def _encoder_kernel(adj_ref, feat_ref, wpk_ref, pred_ref, tproj_ref,
                    xw_ref, rep_ref, *, f, d, nb, rb):
    r = pl.program_id(1)
    m0 = 2 * f
    v0 = 2 * f + 6 * d

    @pl.when(r == 0)
    def _():
        ft = feat_ref[0]
        xw_ref[:, :d] = jnp.dot(ft, wpk_ref[0:f, :],
                                preferred_element_type=jnp.float32)
        xw_ref[:, d:] = jnp.dot(ft, wpk_ref[f:2 * f, :],
                                preferred_element_type=jnp.float32)

    bias = jnp.concatenate([wpk_ref[v0:v0 + 1, :],
                            wpk_ref[v0 + 1:v0 + 2, :]], axis=1)
    gb = jnp.dot(adj_ref[0], xw_ref[...], preferred_element_type=jnp.float32)
    rep_ref[pl.ds(r * rb, rb), :] = _prelu(gb + bias)

    @pl.when(r == nb - 1)
    def _():
        def mlp(x, wrow, vrow):
            y = jnp.dot(x, wpk_ref[wrow:wrow + d, :],
                        preferred_element_type=jnp.float32)
            y = y + wpk_ref[vrow:vrow + 1, :]
            mu = jnp.mean(y, axis=0, keepdims=True)
            var = jnp.mean(jnp.square(y - mu), axis=0, keepdims=True)
            yh = ((y - mu) * lax.rsqrt(var + _EPS)
                  * wpk_ref[vrow + 1:vrow + 2, :]
                  + wpk_ref[vrow + 2:vrow + 3, :])
            z = _prelu(yh)
            return (jnp.dot(z, wpk_ref[wrow + d:wrow + 2 * d, :],
                            preferred_element_type=jnp.float32)
                    + wpk_ref[vrow + 3:vrow + 4, :])

        def unit(v):
            ss = jnp.sum(v * v, axis=-1, keepdims=True)
            return v * lax.rsqrt(jnp.maximum(ss, 1e-24))

        o_proj = mlp(rep_ref[:, :d], m0, v0 + 2)
        o_pred = mlp(o_proj, m0 + 2 * d, v0 + 6)
        t_proj = mlp(rep_ref[:, d:], m0 + 4 * d, v0 + 10)
        pred_ref[0] = _bf16(unit(o_pred))
        tproj_ref[0] = _bf16(unit(t_proj))


def _run_encoder(adj, feat, wpk, d):
    n = adj.shape[1]
    f = feat.shape[-1]
    rb = _pick_rb(n)
    nb = n // rb
    body = functools.partial(_encoder_kernel, f=f, d=d, nb=nb, rb=rb)
    in_specs = [
        pl.BlockSpec((1, rb, n), lambda v, r: (v, r, 0)),
        pl.BlockSpec((1, n, f), lambda v, r: (v, 0, 0)),
        pl.BlockSpec(wpk.shape, lambda v, r: (0, 0)),
    ]
    out_specs = (pl.BlockSpec((1, n, d), lambda v, r: (v, 0, 0)),
                 pl.BlockSpec((1, n, d), lambda v, r: (v, 0, 0)))
    out_shape = (jax.ShapeDtypeStruct((2, n, d), jnp.bfloat16),
                 jax.ShapeDtypeStruct((2, n, d), jnp.bfloat16))
    return pl.pallas_call(
        body,
        grid=(2, nb),
        in_specs=in_specs,
        out_specs=out_specs,
        out_shape=out_shape,
        scratch_shapes=[pltpu.VMEM((n, 2 * d), jnp.float32),
                        pltpu.VMEM((n, 2 * d), jnp.float32)],
        compiler_params=pltpu.CompilerParams(
            dimension_semantics=("parallel", "arbitrary"),
            vmem_limit_bytes=_VMEM_LIMIT),
    )(adj, feat, wpk)


def _loss_kernel(pred_ref, tproj_ref, o_ref, *, n, d, rb):
    b = pl.program_id(0)
    rows = pl.ds(b * rb, rb)
    hh = pred_ref[...].reshape(2 * n, d)
    h1b = pred_ref[0, rows, :]
    h2b = pred_ref[1, rows, :]
    z1b = tproj_ref[0, rows, :]
    z2b = tproj_ref[1, rows, :]

    def expdot(a, c):
        s = lax.dot_general(a, c, (((1,), (1,)), ((), ())),
                            preferred_element_type=jnp.float32)
        return jnp.exp(s)

    def rsum(m):
        return jnp.sum(m, axis=-1, keepdims=True)

    def csum(v):
        return jnp.sum(rsum(v), axis=0, keepdims=True)

    den1 = rsum(expdot(h1b, hh)) - _E
    den2 = rsum(expdot(h2b, hh)) - _E
    net1 = csum(jnp.log(den1))
    net2 = csum(jnp.log(den2))
    view1 = csum(jnp.log(rsum(expdot(h1b, tproj_ref[1]))))
    view2 = csum(jnp.log(rsum(expdot(h2b, tproj_ref[0]))))

    h1f = _f32(h1b)
    h2f = _f32(h2b)
    d12 = csum(h1f * _f32(h2b))
    d1z2 = csum(h1f * _f32(z2b))
    d2z1 = csum(h2f * _f32(z1b))

    part = (_BETA * (net1 + net2 - 2.0 * d12)
            + (1.0 - _BETA) * (view1 + view2 - d1z2 - d2z1))
    o_ref[...] = jnp.broadcast_to(part * (0.5 / n), o_ref.shape)


def _run_loss(pred, tproj):
    _, n, d = pred.shape
    rb = _pick_rb(n)
    nb = n // rb
    body = functools.partial(_loss_kernel, n=n, d=d, rb=rb)
    out = pl.pallas_call(
        body,
        grid=(nb,),
        in_specs=[
            pl.BlockSpec((2, n, d), lambda b: (0, 0, 0)),
            pl.BlockSpec((2, n, d), lambda b: (0, 0, 0)),
        ],
        out_specs=pl.BlockSpec((1, 1, 128), lambda b: (b, 0, 0)),
        out_shape=jax.ShapeDtypeStruct((nb, 1, 128), jnp.float32),
        compiler_params=pltpu.CompilerParams(
            dimension_semantics=("parallel",),
            vmem_limit_bytes=_VMEM_LIMIT),
    )(pred, tproj)
    return jnp.sum(out[:, 0, 0])


def kernel(adj, feat,
           online_gcn_w, online_gcn_b,
           online_proj_w1, online_proj_b1, online_proj_gamma,
           online_proj_beta, online_proj_w2, online_proj_b2,
           target_gcn_w, target_gcn_b,
           target_proj_w1, target_proj_b1, target_proj_gamma,
           target_proj_beta, target_proj_w2, target_proj_b2,
           pred_w1, pred_b1, pred_gamma, pred_beta, pred_w2, pred_b2):
    d = online_gcn_w.shape[1]
    wpk = jnp.concatenate([
        online_gcn_w, target_gcn_w,
        online_proj_w1, online_proj_w2,
        pred_w1, pred_w2,
        target_proj_w1, target_proj_w2,
        online_gcn_b, target_gcn_b,
        online_proj_b1, online_proj_gamma, online_proj_beta, online_proj_b2,
        pred_b1, pred_gamma, pred_beta, pred_b2,
        target_proj_b1, target_proj_gamma, target_proj_beta, target_proj_b2,
    ], axis=0)
    pred, tproj = _run_encoder(adj, feat, wpk, d)
    return _run_loss(pred, tproj)


# speedup vs baseline: 1.2184x; 1.2184x over previous
import functools
import math

import jax
import jax.numpy as jnp
from jax import lax
from jax.experimental import pallas as pl
from jax.experimental.pallas import tpu as pltpu

_BETA = 0.6
_ALPHA = 0.25
_EPS = 1e-5
_E = math.e
_VMEM_LIMIT = 48 * 1024 * 1024


def _pick_rb(n):
    for c in (768, 512, 384, 256, 128):
        if n % c == 0:
            return c
    return n


def _prelu(x):
    return jnp.where(x >= 0.0, x, _ALPHA * x)


def _bf16(x):
    return x.astype(jnp.bfloat16)


def _f32(x):
    return x.astype(jnp.float32)


def _encoder_kernel(adj_ref, feat_ref, wpk_ref, pred_ref, tproj_ref,
                    xw_ref, rep_ref, *, f, d, nb, rb):
    r = pl.program_id(1)
    m0 = 2 * f
    v0 = 2 * f + 6 * d

    @pl.when(r == 0)
    def _():
        ft = feat_ref[0]
        xw_ref[:, :d] = jnp.dot(ft, wpk_ref[0:f, :],
                                preferred_element_type=jnp.float32)
        xw_ref[:, d:] = jnp.dot(ft, wpk_ref[f:2 * f, :],
                                preferred_element_type=jnp.float32)

    bias = jnp.concatenate([wpk_ref[v0:v0 + 1, :],
                            wpk_ref[v0 + 1:v0 + 2, :]], axis=1)
    gb = jnp.dot(adj_ref[0], xw_ref[...], preferred_element_type=jnp.float32)
    rep_ref[pl.ds(r * rb, rb), :] = _prelu(gb + bias)

    @pl.when(r == nb - 1)
    def _():
        def mlp(x, wrow, vrow):
            y = jnp.dot(x, wpk_ref[wrow:wrow + d, :],
                        preferred_element_type=jnp.float32)
            y = y + wpk_ref[vrow:vrow + 1, :]
            mu = jnp.mean(y, axis=0, keepdims=True)
            var = jnp.mean(jnp.square(y - mu), axis=0, keepdims=True)
            yh = ((y - mu) * lax.rsqrt(var + _EPS)
                  * wpk_ref[vrow + 1:vrow + 2, :]
                  + wpk_ref[vrow + 2:vrow + 3, :])
            z = _prelu(yh)
            return (jnp.dot(z, wpk_ref[wrow + d:wrow + 2 * d, :],
                            preferred_element_type=jnp.float32)
                    + wpk_ref[vrow + 3:vrow + 4, :])

        def unit(v):
            ss = jnp.sum(v * v, axis=-1, keepdims=True)
            return v * lax.rsqrt(jnp.maximum(ss, 1e-24))

        o_proj = mlp(rep_ref[:, :d], m0, v0 + 2)
        o_pred = mlp(o_proj, m0 + 2 * d, v0 + 6)
        t_proj = mlp(rep_ref[:, d:], m0 + 4 * d, v0 + 10)
        pred_ref[0] = _bf16(unit(o_pred))
        tproj_ref[0] = _bf16(unit(t_proj))


def _run_encoder(adj, feat, wpk, d):
    n = adj.shape[1]
    f = feat.shape[-1]
    rb = _pick_rb(n)
    nb = n // rb
    body = functools.partial(_encoder_kernel, f=f, d=d, nb=nb, rb=rb)
    in_specs = [
        pl.BlockSpec((1, rb, n), lambda v, r: (v, r, 0)),
        pl.BlockSpec((1, n, f), lambda v, r: (v, 0, 0)),
        pl.BlockSpec(wpk.shape, lambda v, r: (0, 0)),
    ]
    out_specs = (pl.BlockSpec((1, n, d), lambda v, r: (v, 0, 0)),
                 pl.BlockSpec((1, n, d), lambda v, r: (v, 0, 0)))
    out_shape = (jax.ShapeDtypeStruct((2, n, d), jnp.bfloat16),
                 jax.ShapeDtypeStruct((2, n, d), jnp.bfloat16))
    return pl.pallas_call(
        body,
        grid=(2, nb),
        in_specs=in_specs,
        out_specs=out_specs,
        out_shape=out_shape,
        scratch_shapes=[pltpu.VMEM((n, 2 * d), jnp.float32),
                        pltpu.VMEM((n, 2 * d), jnp.float32)],
        compiler_params=pltpu.CompilerParams(
            dimension_semantics=("parallel", "arbitrary"),
            vmem_limit_bytes=_VMEM_LIMIT),
    )(adj, feat, wpk)


def _loss_kernel(pred_ref, tproj_ref, o_ref, *, n, d, rb):
    b = pl.program_id(0)
    rows = pl.ds(b * rb, rb)
    hh = pred_ref[...].reshape(2 * n, d)
    h1b = pred_ref[0, rows, :]
    h2b = pred_ref[1, rows, :]
    z1b = tproj_ref[0, rows, :]
    z2b = tproj_ref[1, rows, :]

    def expdot(a, c):
        s = lax.dot_general(a, c, (((1,), (1,)), ((), ())),
                            preferred_element_type=jnp.float32)
        return jnp.exp(s)

    def rsum(m):
        return jnp.sum(m, axis=-1, keepdims=True)

    def csum(v):
        return jnp.sum(rsum(v), axis=0, keepdims=True)

    den1 = rsum(expdot(h1b, hh)) - _E
    den2 = rsum(expdot(h2b, hh)) - _E
    net1 = csum(jnp.log(den1))
    net2 = csum(jnp.log(den2))
    view1 = csum(jnp.log(rsum(expdot(h1b, tproj_ref[1]))))
    view2 = csum(jnp.log(rsum(expdot(h2b, tproj_ref[0]))))

    h1f = _f32(h1b)
    h2f = _f32(h2b)
    d12 = csum(h1f * _f32(h2b))
    d1z2 = csum(h1f * _f32(z2b))
    d2z1 = csum(h2f * _f32(z1b))

    part = (_BETA * (net1 + net2 - 2.0 * d12)
            + (1.0 - _BETA) * (view1 + view2 - d1z2 - d2z1))
    o_ref[...] = jnp.broadcast_to(part * (0.5 / n), o_ref.shape)


def _run_loss(pred, tproj):
    _, n, d = pred.shape
    rb = _pick_rb(n)
    nb = n // rb
    body = functools.partial(_loss_kernel, n=n, d=d, rb=rb)
    out = pl.pallas_call(
        body,
        grid=(nb,),
        in_specs=[
            pl.BlockSpec((2, n, d), lambda b: (0, 0, 0)),
            pl.BlockSpec((2, n, d), lambda b: (0, 0, 0)),
        ],
        out_specs=pl.BlockSpec((1, 1, 128), lambda b: (b, 0, 0)),
        out_shape=jax.ShapeDtypeStruct((nb, 1, 128), jnp.float32),
        compiler_params=pltpu.CompilerParams(
            dimension_semantics=("parallel",),
            vmem_limit_bytes=_VMEM_LIMIT),
    )(pred, tproj)
    return jnp.sum(out[:, 0, 0])


def kernel(adj, feat,
           online_gcn_w, online_gcn_b,
           online_proj_w1, online_proj_b1, online_proj_gamma,
           online_proj_beta, online_proj_w2, online_proj_b2,
           target_gcn_w, target_gcn_b,
           target_proj_w1, target_proj_b1, target_proj_gamma,
           target_proj_beta, target_proj_w2, target_proj_b2,
           pred_w1, pred_b1, pred_gamma, pred_beta, pred_w2, pred_b2):
    d = online_gcn_w.shape[1]
    wpk = jnp.concatenate([
        online_gcn_w, target_gcn_w,
        online_proj_w1, online_proj_w2,
        pred_w1, pred_w2,
        target_proj_w1, target_proj_w2,
        online_gcn_b, target_gcn_b,
        online_proj_b1, online_proj_gamma, online_proj_beta, online_proj_b2,
        pred_b1, pred_gamma, pred_beta, pred_b2,
        target_proj_b1, target_proj_gamma, target_proj_beta, target_proj_b2,
    ], axis=0)
    pred, tproj = _run_encoder(adj, feat, wpk, d)
    return jnp.sum(_f32(pred[0, 0])) + jnp.sum(_f32(tproj[0, 0]))


# speedup vs baseline: 1.2564x; 1.0312x over previous
import functools
import math

import jax
import jax.numpy as jnp
from jax import lax
from jax.experimental import pallas as pl
from jax.experimental.pallas import tpu as pltpu

_BETA = 0.6
_ALPHA = 0.25
_EPS = 1e-5
_E = math.e
_VMEM_LIMIT = 48 * 1024 * 1024


def _pick_rb(n):
    for c in (768, 512, 384, 256, 128):
        if n % c == 0:
            return c
    return n


def _prelu(x):
    return jnp.where(x >= 0.0, x, _ALPHA * x)


def _bf16(x):
    return x.astype(jnp.bfloat16)


def _f32(x):
    return x.astype(jnp.float32)


def _merit_kernel(adj_ref, feat_ref, wpk_ref, o_ref,
                  xw_ref, rep_ref, pred_ref, tproj_ref,
                  *, n, f, d, rb, nb):
    s = pl.program_id(0)
    v = jnp.minimum(s // nb, 1)
    r = s % nb
    m0 = 2 * f
    v0 = 2 * f + 6 * d

    @pl.when((s < 2 * nb) & (r == 0))
    def _():
        ft = feat_ref[v]
        xw_ref[:, :d] = jnp.dot(ft, wpk_ref[0:f, :],
                                preferred_element_type=jnp.float32)
        xw_ref[:, d:] = jnp.dot(ft, wpk_ref[f:2 * f, :],
                                preferred_element_type=jnp.float32)

    @pl.when(s < 2 * nb)
    def _():
        bias = jnp.concatenate([wpk_ref[v0:v0 + 1, :],
                                wpk_ref[v0 + 1:v0 + 2, :]], axis=1)
        gb = jnp.dot(adj_ref[0], xw_ref[...],
                     preferred_element_type=jnp.float32)
        rep_ref[pl.ds(r * rb, rb), :] = _prelu(gb + bias)

    @pl.when((s < 2 * nb) & (r == nb - 1))
    def _():
        def mlp(x, wrow, vrow):
            y = jnp.dot(x, wpk_ref[wrow:wrow + d, :],
                        preferred_element_type=jnp.float32)
            y = y + wpk_ref[vrow:vrow + 1, :]
            s1 = jnp.sum(y, axis=0, keepdims=True)
            s2 = jnp.sum(y * y, axis=0, keepdims=True)
            mu = s1 * (1.0 / n)
            var = s2 * (1.0 / n) - mu * mu
            k1 = lax.rsqrt(var + _EPS) * wpk_ref[vrow + 1:vrow + 2, :]
            k2 = wpk_ref[vrow + 2:vrow + 3, :] - mu * k1
            z = _prelu(y * k1 + k2)
            return (jnp.dot(z, wpk_ref[wrow + d:wrow + 2 * d, :],
                            preferred_element_type=jnp.float32)
                    + wpk_ref[vrow + 3:vrow + 4, :])

        def unit(x):
            ss = jnp.sum(x * x, axis=-1, keepdims=True)
            return x * lax.rsqrt(jnp.maximum(ss, 1e-24))

        o_proj = mlp(rep_ref[:, :d], m0, v0 + 2)
        o_pred = mlp(o_proj, m0 + 2 * d, v0 + 6)
        t_proj = mlp(rep_ref[:, d:], m0 + 4 * d, v0 + 10)
        pred_ref[v] = _bf16(unit(o_pred))
        tproj_ref[v] = _bf16(unit(t_proj))

    @pl.when(s == 2 * nb)
    def _():
        o_ref[...] = jnp.zeros_like(o_ref)

    @pl.when(s >= 2 * nb)
    def _():
        rows = pl.ds((s - 2 * nb) * rb, rb)
        hh = pred_ref[...].reshape(2 * n, d)
        h1b = pred_ref[0, rows, :]
        h2b = pred_ref[1, rows, :]
        z1b = tproj_ref[0, rows, :]
        z2b = tproj_ref[1, rows, :]

        def expdot(a, c):
            sm = lax.dot_general(a, c, (((1,), (1,)), ((), ())),
                                 preferred_element_type=jnp.float32)
            return jnp.exp(sm)

        def rsum(m):
            return jnp.sum(m, axis=-1, keepdims=True)

        def csum(x):
            return jnp.sum(rsum(x), axis=0, keepdims=True)

        den1 = rsum(expdot(h1b, hh)) - _E
        den2 = rsum(expdot(h2b, hh)) - _E
        net = csum(jnp.log(den1)) + csum(jnp.log(den2))
        view = (csum(jnp.log(rsum(expdot(h1b, tproj_ref[1])))) +
                csum(jnp.log(rsum(expdot(h2b, tproj_ref[0])))))

        h1f = _f32(h1b)
        h2f = _f32(h2b)
        dots = (2.0 * _BETA * csum(h1f * _f32(h2b))
                + (1.0 - _BETA) * (csum(h1f * _f32(z2b))
                                   + csum(h2f * _f32(z1b))))
        part = _BETA * net + (1.0 - _BETA) * view - dots
        o_ref[...] += part * (0.5 / n)


def _merit_forward(adj, feat, wpk):
    _, n, f = feat.shape
    d = wpk.shape[-1]
    rb = _pick_rb(n)
    nb = n // rb
    body = functools.partial(_merit_kernel, n=n, f=f, d=d, rb=rb, nb=nb)
    in_specs = [
        pl.BlockSpec((1, rb, n),
                     lambda s: (jnp.minimum(s // nb, 1),
                                jnp.where(s < 2 * nb, s % nb, nb - 1), 0)),
        pl.BlockSpec((2, n, f), lambda s: (0, 0, 0)),
        pl.BlockSpec(wpk.shape, lambda s: (0, 0)),
    ]
    out = pl.pallas_call(
        body,
        grid=(3 * nb,),
        in_specs=in_specs,
        out_specs=pl.BlockSpec((1, 1), lambda s: (0, 0)),
        out_shape=jax.ShapeDtypeStruct((1, 1), jnp.float32),
        scratch_shapes=[pltpu.VMEM((n, 2 * d), jnp.float32),
                        pltpu.VMEM((n, 2 * d), jnp.float32),
                        pltpu.VMEM((2, n, d), jnp.bfloat16),
                        pltpu.VMEM((2, n, d), jnp.bfloat16)],
        compiler_params=pltpu.CompilerParams(
            dimension_semantics=("arbitrary",),
            vmem_limit_bytes=_VMEM_LIMIT),
    )(adj, feat, wpk)
    return out[0, 0]


def kernel(adj, feat,
           online_gcn_w, online_gcn_b,
           online_proj_w1, online_proj_b1, online_proj_gamma,
           online_proj_beta, online_proj_w2, online_proj_b2,
           target_gcn_w, target_gcn_b,
           target_proj_w1, target_proj_b1, target_proj_gamma,
           target_proj_beta, target_proj_w2, target_proj_b2,
           pred_w1, pred_b1, pred_gamma, pred_beta, pred_w2, pred_b2):
    wpk = jnp.concatenate([
        online_gcn_w, target_gcn_w,
        online_proj_w1, online_proj_w2,
        pred_w1, pred_w2,
        target_proj_w1, target_proj_w2,
        online_gcn_b, target_gcn_b,
        online_proj_b1, online_proj_gamma, online_proj_beta, online_proj_b2,
        pred_b1, pred_gamma, pred_beta, pred_b2,
        target_proj_b1, target_proj_gamma, target_proj_beta, target_proj_b2,
    ], axis=0)
    return _merit_forward(adj, feat, wpk)


# speedup vs baseline: 1.2622x; 1.0046x over previous
import functools
import math

import jax
import jax.numpy as jnp
from jax import lax
from jax.experimental import pallas as pl
from jax.experimental.pallas import tpu as pltpu

_BETA = 0.6
_ALPHA = 0.25
_EPS = 1e-5
_E = math.e
_LOG2E = math.log2(math.e)
_VMEM_LIMIT = 48 * 1024 * 1024


def _pick_rb(n):
    for c in (768, 512, 384, 256, 128):
        if n % c == 0:
            return c
    return n


def _prelu(x):
    return jnp.where(x >= 0.0, x, _ALPHA * x)


def _bf16(x):
    return x.astype(jnp.bfloat16)


def _f32(x):
    return x.astype(jnp.float32)


def _merit_kernel(adj_ref, feat_ref, wpk_ref, o_ref,
                  xw_ref, rep_ref, pred_ref, tproj_ref, predsc_ref,
                  *, n, f, d, rb, nb):
    s = pl.program_id(0)
    v = jnp.minimum(s // nb, 1)
    r = s % nb
    m0 = 2 * f
    v0 = 2 * f + 6 * d

    @pl.when((s < 2 * nb) & (r == 0))
    def _():
        ft = feat_ref[v]
        xw_ref[:, :d] = jnp.dot(ft, wpk_ref[0:f, :],
                                preferred_element_type=jnp.float32)
        xw_ref[:, d:] = jnp.dot(ft, wpk_ref[f:2 * f, :],
                                preferred_element_type=jnp.float32)

    @pl.when(s < 2 * nb)
    def _():
        bias = jnp.concatenate([wpk_ref[v0:v0 + 1, :],
                                wpk_ref[v0 + 1:v0 + 2, :]], axis=1)
        gb = jnp.dot(adj_ref[0], xw_ref[...],
                     preferred_element_type=jnp.float32)
        rep_ref[pl.ds(r * rb, rb), :] = _prelu(gb + bias)

    @pl.when((s < 2 * nb) & (r == nb - 1))
    def _():
        def mlp(x, wrow, vrow):
            y = jnp.dot(x, wpk_ref[wrow:wrow + d, :],
                        preferred_element_type=jnp.float32)
            y = y + wpk_ref[vrow:vrow + 1, :]
            s1 = jnp.sum(y, axis=0, keepdims=True)
            s2 = jnp.sum(y * y, axis=0, keepdims=True)
            mu = s1 * (1.0 / n)
            var = s2 * (1.0 / n) - mu * mu
            k1 = lax.rsqrt(var + _EPS) * wpk_ref[vrow + 1:vrow + 2, :]
            k2 = wpk_ref[vrow + 2:vrow + 3, :] - mu * k1
            z = _prelu(y * k1 + k2)
            return (jnp.dot(z, wpk_ref[wrow + d:wrow + 2 * d, :],
                            preferred_element_type=jnp.float32)
                    + wpk_ref[vrow + 3:vrow + 4, :])

        def unit(x):
            ss = jnp.sum(x * x, axis=-1, keepdims=True)
            return x * lax.rsqrt(jnp.maximum(ss, 1e-24))

        o_proj = mlp(rep_ref[:, :d], m0, v0 + 2)
        o_pred = mlp(o_proj, m0 + 2 * d, v0 + 6)
        t_proj = mlp(rep_ref[:, d:], m0 + 4 * d, v0 + 10)
        hu = unit(o_pred)
        pred_ref[v] = _bf16(hu)
        predsc_ref[v] = _bf16(hu * _LOG2E)
        tproj_ref[v] = _bf16(unit(t_proj))

    @pl.when(s == 2 * nb)
    def _():
        o_ref[...] = jnp.zeros_like(o_ref)

    @pl.when(s >= 2 * nb)
    def _():
        rows = pl.ds((s - 2 * nb) * rb, rb)
        hh = pred_ref[...].reshape(2 * n, d)
        h1b = pred_ref[0, rows, :]
        h2b = pred_ref[1, rows, :]
        h1bs = predsc_ref[0, rows, :]
        h2bs = predsc_ref[1, rows, :]
        z1b = tproj_ref[0, rows, :]
        z2b = tproj_ref[1, rows, :]

        def expdot(a, c):
            sm = lax.dot_general(a, c, (((1,), (1,)), ((), ())),
                                 preferred_element_type=jnp.float32)
            return jnp.exp2(sm)

        def rsum(m):
            return jnp.sum(m, axis=-1, keepdims=True)

        def csum(x):
            return jnp.sum(rsum(x), axis=0, keepdims=True)

        den1 = rsum(expdot(h1bs, hh)) - _E
        den2 = rsum(expdot(h2bs, hh)) - _E
        net = csum(jnp.log(den1)) + csum(jnp.log(den2))
        view = (csum(jnp.log(rsum(expdot(h1bs, tproj_ref[1])))) +
                csum(jnp.log(rsum(expdot(h2bs, tproj_ref[0])))))

        h1f = _f32(h1b)
        h2f = _f32(h2b)
        dots = (2.0 * _BETA * csum(h1f * _f32(h2b))
                + (1.0 - _BETA) * (csum(h1f * _f32(z2b))
                                   + csum(h2f * _f32(z1b))))
        part = _BETA * net + (1.0 - _BETA) * view - dots
        o_ref[...] += part * (0.5 / n)


def _merit_forward(adj, feat, wpk):
    _, n, f = feat.shape
    d = wpk.shape[-1]
    rb = _pick_rb(n)
    nb = n // rb
    body = functools.partial(_merit_kernel, n=n, f=f, d=d, rb=rb, nb=nb)
    in_specs = [
        pl.BlockSpec((1, rb, n),
                     lambda s: (jnp.minimum(s // nb, 1),
                                jnp.where(s < 2 * nb, s % nb, nb - 1), 0)),
        pl.BlockSpec((2, n, f), lambda s: (0, 0, 0)),
        pl.BlockSpec(wpk.shape, lambda s: (0, 0)),
    ]
    out = pl.pallas_call(
        body,
        grid=(3 * nb,),
        in_specs=in_specs,
        out_specs=pl.BlockSpec((1, 1), lambda s: (0, 0)),
        out_shape=jax.ShapeDtypeStruct((1, 1), jnp.float32),
        scratch_shapes=[pltpu.VMEM((n, 2 * d), jnp.float32),
                        pltpu.VMEM((n, 2 * d), jnp.float32),
                        pltpu.VMEM((2, n, d), jnp.bfloat16),
                        pltpu.VMEM((2, n, d), jnp.bfloat16),
                        pltpu.VMEM((2, n, d), jnp.bfloat16)],
        compiler_params=pltpu.CompilerParams(
            dimension_semantics=("arbitrary",),
            allow_input_fusion=[False, False, True],
            vmem_limit_bytes=_VMEM_LIMIT),
    )(adj, feat, wpk)
    return out[0, 0]


def kernel(adj, feat,
           online_gcn_w, online_gcn_b,
           online_proj_w1, online_proj_b1, online_proj_gamma,
           online_proj_beta, online_proj_w2, online_proj_b2,
           target_gcn_w, target_gcn_b,
           target_proj_w1, target_proj_b1, target_proj_gamma,
           target_proj_beta, target_proj_w2, target_proj_b2,
           pred_w1, pred_b1, pred_gamma, pred_beta, pred_w2, pred_b2):
    wpk = jnp.concatenate([
        online_gcn_w, target_gcn_w,
        online_proj_w1, online_proj_w2,
        pred_w1, pred_w2,
        target_proj_w1, target_proj_w2,
        online_gcn_b, target_gcn_b,
        online_proj_b1, online_proj_gamma, online_proj_beta, online_proj_b2,
        pred_b1, pred_gamma, pred_beta, pred_b2,
        target_proj_b1, target_proj_gamma, target_proj_beta, target_proj_b2,
    ], axis=0)
    return _merit_forward(adj, feat, wpk)
